# baseline (speedup 1.0000x reference)
import functools

import jax
import jax.numpy as jnp
from jax import lax
from jax.experimental import pallas as pl
from jax.experimental.pallas import tpu as pltpu
from jax.experimental.pallas import tpu_sc as plsc

N = 10000
E = 320000
D = 128

NC = 2
NS = 16
NW = NC * NS
CH = 128
NCHUNK = 80
NPH = 2
PH = NCHUNK // NPH
E_PAD = NW * NCHUNK * CH
N_PAD = 10240
ROWS_PT = N_PAD // NS

def _sc_body(y_hbm, src_hbm, dst_hbm, ones_hbm,
             part_hbm, degp_hbm,
             acc_sh, deg_sh, src_all, dst_all, rows0, rows1, ones_v,
             sem0, sem1, with_deg):
    c = lax.axis_index("c")
    s = lax.axis_index("s")
    wid = s * NC + c
    r0 = s * ROWS_PT

    rows = [rows0, rows1]
    sems = [sem0, sem1]
    c0 = pl.multiple_of(wid * NCHUNK, 8)

    def _zr(i, _):
        for jj in range(D // 16):
            rows0[i, pl.ds(jj * 16, 16)] = jnp.zeros((16,), jnp.float32)
        return 0

    lax.fori_loop(0, CH, _zr, 0)
    for k in range(ROWS_PT // CH):
        pltpu.sync_copy(rows0, acc_sh.at[pl.ds(r0 + k * CH, CH)])
    if with_deg:
        for k in range(ROWS_PT // CH):
            pltpu.sync_copy(rows0.at[0], deg_sh.at[pl.ds(r0 + k * CH, CH)])
        pltpu.sync_copy(ones_hbm, ones_v)
    plsc.subcore_barrier()

    for p in range(NPH):
        pltpu.sync_copy(src_hbm.at[pl.ds(c0 + p * PH, PH)], src_all)
        pltpu.sync_copy(dst_hbm.at[pl.ds(c0 + p * PH, PH)], dst_all)
        pltpu.async_copy(y_hbm.at[src_all.at[0]], rows[0], sems[0])
        pltpu.async_copy(y_hbm.at[src_all.at[1]], rows[1], sems[1])


        def _pair_body(j, _):
            for b in range(2):
                i = j * 2 + b
                pltpu.make_async_copy(
                    y_hbm.at[src_all.at[i]], rows[b], sems[b]).wait()
                pltpu.sync_copy(rows[b], acc_sh.at[dst_all.at[i]], add=True)
                if with_deg:
                    pltpu.sync_copy(ones_v, deg_sh.at[dst_all.at[i]], add=True)

                @pl.when(i + 2 < PH)
                def _():
                    pltpu.async_copy(y_hbm.at[src_all.at[i + 2]], rows[b], sems[b])
            return 0

        lax.fori_loop(0, PH // 2, _pair_body, 0)
    plsc.subcore_barrier()

    o0 = c * N_PAD + r0
    pltpu.sync_copy(acc_sh.at[pl.ds(r0, ROWS_PT)], part_hbm.at[pl.ds(o0, ROWS_PT)])
    if with_deg:
        pltpu.sync_copy(deg_sh.at[pl.ds(r0, ROWS_PT)], degp_hbm.at[pl.ds(o0, ROWS_PT)])


@functools.cache
def _sc_kernels():
    mesh = plsc.VectorSubcoreMesh(core_axis_name="c", subcore_axis_name="s")

    @functools.partial(
        pl.kernel,
        out_type=[jax.ShapeDtypeStruct((2 * N_PAD, D), jnp.float32),
                  jax.ShapeDtypeStruct((2 * N_PAD,), jnp.float32)],
        mesh=mesh,
        scratch_types=[
            pltpu.VMEM_SHARED((N_PAD, D), jnp.float32),
            pltpu.VMEM_SHARED((N_PAD,), jnp.float32),
            pltpu.VMEM((PH, CH), jnp.int32),
            pltpu.VMEM((PH, CH), jnp.int32),
            pltpu.VMEM((CH, D), jnp.float32),
            pltpu.VMEM((CH, D), jnp.float32),
            pltpu.VMEM((CH,), jnp.float32),
            pltpu.SemaphoreType.DMA,
            pltpu.SemaphoreType.DMA,
        ],
    )
    def sc_agg_deg(y_hbm, src_hbm, dst_hbm, ones_hbm,
                   part_hbm, degp_hbm,
                   acc_sh, deg_sh, src_all, dst_all, rows0, rows1, ones_v,
                   sem0, sem1):
        _sc_body(y_hbm, src_hbm, dst_hbm, ones_hbm,
                 part_hbm, degp_hbm,
                 acc_sh, deg_sh, src_all, dst_all, rows0, rows1, ones_v,
                 sem0, sem1, True)

    @functools.partial(
        pl.kernel,
        out_type=[jax.ShapeDtypeStruct((2 * N_PAD, D), jnp.float32)],
        mesh=mesh,
        scratch_types=[
            pltpu.VMEM_SHARED((N_PAD, D), jnp.float32),
            pltpu.VMEM((PH, CH), jnp.int32),
            pltpu.VMEM((PH, CH), jnp.int32),
            pltpu.VMEM((CH, D), jnp.float32),
            pltpu.VMEM((CH, D), jnp.float32),
            pltpu.SemaphoreType.DMA,
            pltpu.SemaphoreType.DMA,
        ],
    )
    def sc_agg(y_hbm, src_hbm, dst_hbm,
               part_hbm,
               acc_sh, src_all, dst_all, rows0, rows1, sem0, sem1):
        _sc_body(y_hbm, src_hbm, dst_hbm, None, part_hbm, None,
                 acc_sh, None, src_all, dst_all, rows0, rows1, None,
                 sem0, sem1, False)

    return sc_agg_deg, sc_agg


BN = 1000
BNP = 1024


def _comb_body(p_ref, dg0_ref, dg1_ref, x_ref, b_ref, wm1_ref, wr1_ref,
               wr2_ref, h_ref, r2_ref, invb_ref):
    invd_row = 1.0 / jnp.maximum(dg0_ref[...] + dg1_ref[...], 1.0)
    invd = jnp.swapaxes(invd_row, 0, 1)
    agg = (p_ref[0] + p_ref[1]) * invd
    h = jnp.maximum(
        jnp.dot(agg, wm1_ref[...], preferred_element_type=jnp.float32)
        + jnp.dot(x_ref[...], wr1_ref[...], preferred_element_type=jnp.float32)
        + b_ref[...], 0.0)
    h_ref[...] = h
    r2_ref[...] = jnp.dot(h, wr2_ref[...], preferred_element_type=jnp.float32)
    invb_ref[...] = jnp.broadcast_to(invd, (BNP, D))


_comb = pl.pallas_call(
    _comb_body,
    grid=(N_PAD // BNP,),
    in_specs=[pl.BlockSpec((2, BNP, D), lambda i: (0, i, 0)),
              pl.BlockSpec((1, BNP), lambda i: (0, i)),
              pl.BlockSpec((1, BNP), lambda i: (0, i)),
              pl.BlockSpec((BNP, D), lambda i: (i, 0)),
              pl.BlockSpec((1, D), lambda i: (0, 0)),
              pl.BlockSpec((D, D), lambda i: (0, 0)),
              pl.BlockSpec((D, D), lambda i: (0, 0)),
              pl.BlockSpec((D, D), lambda i: (0, 0))],
    out_specs=[pl.BlockSpec((BNP, D), lambda i: (i, 0)),
               pl.BlockSpec((BNP, D), lambda i: (i, 0)),
               pl.BlockSpec((BNP, D), lambda i: (i, 0))],
    out_shape=[jax.ShapeDtypeStruct((N_PAD, D), jnp.float32),
               jax.ShapeDtypeStruct((N_PAD, D), jnp.float32),
               jax.ShapeDtypeStruct((N_PAD, D), jnp.float32)],
)


def _fin_body(p_ref, invb_ref, r_ref, b_ref, wm2_ref, o_ref):
    agg = (p_ref[0] + p_ref[1]) * invb_ref[...]
    o_ref[...] = jnp.maximum(
        jnp.dot(agg, wm2_ref[...], preferred_element_type=jnp.float32)
        + r_ref[...] + b_ref[...], 0.0)


_fin = pl.pallas_call(
    _fin_body,
    grid=(N // BN,),
    in_specs=[pl.BlockSpec((2, BN, D), lambda i: (0, i, 0)),
              pl.BlockSpec((BN, D), lambda i: (i, 0)),
              pl.BlockSpec((BN, D), lambda i: (i, 0)),
              pl.BlockSpec((1, D), lambda i: (0, 0)),
              pl.BlockSpec((D, D), lambda i: (0, 0))],
    out_specs=pl.BlockSpec((BN, D), lambda i: (i, 0)),
    out_shape=jax.ShapeDtypeStruct((N, D), jnp.float32),
)


def kernel(x, edge_index, W_msg1, W_root1, b1, W_msg2, W_root2, b2):
    pad_rows = (E_PAD - E) // CH
    e3 = edge_index.reshape(2, E // CH, CH)
    src2 = e3[0]
    dst2 = e3[1]
    pad_iota = jax.lax.broadcasted_iota(jnp.int32, (pad_rows, CH), 1)
    src = jnp.concatenate([src2, pad_iota], axis=0)
    dst = jnp.concatenate([dst2, N + pad_iota], axis=0)
    ones = jnp.ones((CH,), jnp.float32)
    x_pad = jnp.pad(x, ((0, N_PAD - N), (0, 0)))

    sc_agg_deg, sc_agg = _sc_kernels()
    part1, degp1 = sc_agg_deg(x, src, dst, ones)
    p1 = part1.reshape(2, N_PAD, D)
    dg0 = degp1[:N_PAD].reshape(1, N_PAD)
    dg1 = degp1[N_PAD:].reshape(1, N_PAD)
    h, r2, invb = _comb(p1, dg0, dg1, x_pad, b1.reshape(1, D),
                        W_msg1, W_root1, W_root2)
    (part2,) = sc_agg(h, src, dst)
    out = _fin(part2.reshape(2, N_PAD, D), invb, r2, b2.reshape(1, D), W_msg2)
    return out


# speedup vs baseline: 1.0437x; 1.0437x over previous
import functools

import jax
import jax.numpy as jnp
from jax import lax
from jax.experimental import pallas as pl
from jax.experimental.pallas import tpu as pltpu
from jax.experimental.pallas import tpu_sc as plsc

N = 10000
E = 320000
D = 128

NC = 2
NS = 16
NW = NC * NS
CH = 128
NCHUNK = 80
NPH = 2
PH = NCHUNK // NPH
E_PAD = NW * NCHUNK * CH
N_PAD = 10240
ROWS_PT = N_PAD // NS

def _sc_body(y_hbm, src_hbm, dst_hbm, ones_hbm,
             part_hbm, degp_hbm,
             acc_sh, deg_sh, src_all, dst_all, rows0, rows1, ones_v,
             sem0, sem1, with_deg):
    c = lax.axis_index("c")
    s = lax.axis_index("s")
    wid = s * NC + c
    r0 = s * ROWS_PT

    rows = [rows0, rows1]
    sems = [sem0, sem1]
    c0 = pl.multiple_of(wid * NCHUNK, 8)

    def _zr(i, _):
        for jj in range(D // 16):
            rows0[i, pl.ds(jj * 16, 16)] = jnp.zeros((16,), jnp.float32)
        return 0

    lax.fori_loop(0, CH, _zr, 0)
    for k in range(ROWS_PT // CH):
        pltpu.sync_copy(rows0, acc_sh.at[pl.ds(r0 + k * CH, CH)])
    if with_deg:
        for k in range(ROWS_PT // CH):
            pltpu.sync_copy(rows0.at[0], deg_sh.at[pl.ds(r0 + k * CH, CH)])
        pltpu.sync_copy(ones_hbm, ones_v)
    plsc.subcore_barrier()

    for p in range(NPH):
        pltpu.sync_copy(src_hbm.at[pl.ds(c0 + p * PH, PH)], src_all)
        pltpu.sync_copy(dst_hbm.at[pl.ds(c0 + p * PH, PH)], dst_all)
        pltpu.async_copy(y_hbm.at[src_all.at[0]], rows[0], sems[0])
        pltpu.async_copy(y_hbm.at[src_all.at[1]], rows[1], sems[1])


        def _pair_body(j, _):
            for b in range(2):
                i = j * 2 + b
                pltpu.make_async_copy(
                    y_hbm.at[src_all.at[i]], rows[b], sems[b]).wait()
                pltpu.sync_copy(rows[b], acc_sh.at[dst_all.at[i]], add=True)
                if with_deg:
                    pltpu.sync_copy(ones_v, deg_sh.at[dst_all.at[i]], add=True)

                @pl.when(i + 2 < PH)
                def _():
                    pltpu.async_copy(y_hbm.at[src_all.at[i + 2]], rows[b], sems[b])
            return 0

        lax.fori_loop(0, PH // 2, _pair_body, 0)
    plsc.subcore_barrier()

    o0 = c * N_PAD + r0
    pltpu.sync_copy(acc_sh.at[pl.ds(r0, ROWS_PT)], part_hbm.at[pl.ds(o0, ROWS_PT)])
    if with_deg:
        pltpu.sync_copy(deg_sh.at[pl.ds(r0, ROWS_PT)], degp_hbm.at[pl.ds(o0, ROWS_PT)])


@functools.cache
def _sc_kernels():
    mesh = plsc.VectorSubcoreMesh(core_axis_name="c", subcore_axis_name="s")

    @functools.partial(
        pl.kernel,
        out_type=[jax.ShapeDtypeStruct((2 * N_PAD, D), jnp.float32),
                  jax.ShapeDtypeStruct((2 * N_PAD,), jnp.float32)],
        mesh=mesh,
        scratch_types=[
            pltpu.VMEM_SHARED((N_PAD, D), jnp.float32),
            pltpu.VMEM_SHARED((N_PAD,), jnp.float32),
            pltpu.VMEM((PH, CH), jnp.int32),
            pltpu.VMEM((PH, CH), jnp.int32),
            pltpu.VMEM((CH, D), jnp.float32),
            pltpu.VMEM((CH, D), jnp.float32),
            pltpu.VMEM((CH,), jnp.float32),
            pltpu.SemaphoreType.DMA,
            pltpu.SemaphoreType.DMA,
        ],
    )
    def sc_agg_deg(y_hbm, src_hbm, dst_hbm, ones_hbm,
                   part_hbm, degp_hbm,
                   acc_sh, deg_sh, src_all, dst_all, rows0, rows1, ones_v,
                   sem0, sem1):
        _sc_body(y_hbm, src_hbm, dst_hbm, ones_hbm,
                 part_hbm, degp_hbm,
                 acc_sh, deg_sh, src_all, dst_all, rows0, rows1, ones_v,
                 sem0, sem1, True)

    @functools.partial(
        pl.kernel,
        out_type=[jax.ShapeDtypeStruct((2 * N_PAD, D), jnp.float32)],
        mesh=mesh,
        scratch_types=[
            pltpu.VMEM_SHARED((N_PAD, D), jnp.float32),
            pltpu.VMEM((PH, CH), jnp.int32),
            pltpu.VMEM((PH, CH), jnp.int32),
            pltpu.VMEM((CH, D), jnp.float32),
            pltpu.VMEM((CH, D), jnp.float32),
            pltpu.SemaphoreType.DMA,
            pltpu.SemaphoreType.DMA,
        ],
    )
    def sc_agg(y_hbm, src_hbm, dst_hbm,
               part_hbm,
               acc_sh, src_all, dst_all, rows0, rows1, sem0, sem1):
        _sc_body(y_hbm, src_hbm, dst_hbm, None, part_hbm, None,
                 acc_sh, None, src_all, dst_all, rows0, rows1, None,
                 sem0, sem1, False)

    return sc_agg_deg, sc_agg


BN = 1000
BNP = 1024


def _comb_body(p_ref, dg0_ref, dg1_ref, x_ref, b_ref, wm1_ref, wr1_ref,
               wr2_ref, h_ref, r2_ref, invb_ref):
    invd_row = 1.0 / jnp.maximum(dg0_ref[...] + dg1_ref[...], 1.0)
    invd = jnp.swapaxes(invd_row, 0, 1)
    agg = (p_ref[0] + p_ref[1]) * invd
    h = jnp.maximum(
        jnp.dot(agg, wm1_ref[...], preferred_element_type=jnp.float32)
        + jnp.dot(x_ref[...], wr1_ref[...], preferred_element_type=jnp.float32)
        + b_ref[...], 0.0)
    h_ref[...] = h
    r2_ref[...] = jnp.dot(h, wr2_ref[...], preferred_element_type=jnp.float32)
    invb_ref[...] = jnp.broadcast_to(invd, (BNP, D))


_comb = pl.pallas_call(
    _comb_body,
    grid=(N_PAD // BNP,),
    in_specs=[pl.BlockSpec((2, BNP, D), lambda i: (0, i, 0)),
              pl.BlockSpec((1, BNP), lambda i: (0, i)),
              pl.BlockSpec((1, BNP), lambda i: (0, i)),
              pl.BlockSpec((BNP, D), lambda i: (i, 0)),
              pl.BlockSpec((1, D), lambda i: (0, 0)),
              pl.BlockSpec((D, D), lambda i: (0, 0)),
              pl.BlockSpec((D, D), lambda i: (0, 0)),
              pl.BlockSpec((D, D), lambda i: (0, 0))],
    out_specs=[pl.BlockSpec((BNP, D), lambda i: (i, 0)),
               pl.BlockSpec((BNP, D), lambda i: (i, 0)),
               pl.BlockSpec((BNP, D), lambda i: (i, 0))],
    out_shape=[jax.ShapeDtypeStruct((N_PAD, D), jnp.float32),
               jax.ShapeDtypeStruct((N_PAD, D), jnp.float32),
               jax.ShapeDtypeStruct((N_PAD, D), jnp.float32)],
)


def _fin_body(p_ref, invb_ref, r_ref, b_ref, wm2_ref, o_ref):
    agg = (p_ref[0] + p_ref[1]) * invb_ref[...]
    o_ref[...] = jnp.maximum(
        jnp.dot(agg, wm2_ref[...], preferred_element_type=jnp.float32)
        + r_ref[...] + b_ref[...], 0.0)


_fin = pl.pallas_call(
    _fin_body,
    grid=(N // BN,),
    in_specs=[pl.BlockSpec((2, BN, D), lambda i: (0, i, 0)),
              pl.BlockSpec((BN, D), lambda i: (i, 0)),
              pl.BlockSpec((BN, D), lambda i: (i, 0)),
              pl.BlockSpec((1, D), lambda i: (0, 0)),
              pl.BlockSpec((D, D), lambda i: (0, 0))],
    out_specs=pl.BlockSpec((BN, D), lambda i: (i, 0)),
    out_shape=jax.ShapeDtypeStruct((N, D), jnp.float32),
)


def _esplit_body(e_ref, s_ref, d_ref):
    s_ref[...] = e_ref[0].reshape(E // CH, CH)
    d_ref[...] = e_ref[1].reshape(E // CH, CH)


_esplit = pl.pallas_call(
    _esplit_body,
    out_shape=[jax.ShapeDtypeStruct((E // CH, CH), jnp.int32),
               jax.ShapeDtypeStruct((E // CH, CH), jnp.int32)],
)


def kernel(x, edge_index, W_msg1, W_root1, b1, W_msg2, W_root2, b2):
    pad_rows = (E_PAD - E) // CH
    src2, dst2 = _esplit(edge_index)
    pad_iota = jax.lax.broadcasted_iota(jnp.int32, (pad_rows, CH), 1)
    src = jnp.concatenate([src2, pad_iota], axis=0)
    dst = jnp.concatenate([dst2, N + pad_iota], axis=0)
    ones = jnp.ones((CH,), jnp.float32)
    x_pad = jnp.pad(x, ((0, N_PAD - N), (0, 0)))

    sc_agg_deg, sc_agg = _sc_kernels()
    part1, degp1 = sc_agg_deg(x, src, dst, ones)
    p1 = part1.reshape(2, N_PAD, D)
    dg0 = degp1[:N_PAD].reshape(1, N_PAD)
    dg1 = degp1[N_PAD:].reshape(1, N_PAD)
    h, r2, invb = _comb(p1, dg0, dg1, x_pad, b1.reshape(1, D),
                        W_msg1, W_root1, W_root2)
    (part2,) = sc_agg(h, src, dst)
    out = _fin(part2.reshape(2, N_PAD, D), invb, r2, b2.reshape(1, D), W_msg2)
    return out
